# baseline (speedup 1.0000x reference)
import functools

import jax
import jax.numpy as jnp
from jax import lax
from jax.experimental import pallas as pl
from jax.experimental.pallas import tpu as pltpu
from jax.experimental.pallas import tpu_sc as plsc

GRP = 8
NBUF = 4
LAG = 2
SPLITS = (2048, 2048, 4096, 8192)


def _sc_gather(ids, table):
    batch, hist = ids.shape
    d = table.shape[1]

    info = plsc.get_sparse_core_info()
    nc, ns = info.num_cores, info.num_subcores
    nw = nc * ns
    per_b = batch // nw
    rounds = per_b // GRP

    mesh = plsc.VectorSubcoreMesh(core_axis_name="c", subcore_axis_name="s")

    @functools.partial(
        pl.kernel,
        mesh=mesh,
        compiler_params=pltpu.CompilerParams(use_tc_tiling_on_sc=False),
        out_type=jax.ShapeDtypeStruct((batch, hist, d), table.dtype),
        scratch_types=[
            pltpu.VMEM((per_b, hist), jnp.int32),
            pltpu.VMEM((NBUF, GRP, hist, d), jnp.float32),
            pltpu.SemaphoreType.DMA((NBUF,)),
            pltpu.SemaphoreType.DMA((NBUF,)),
        ],
    )
    def k(table_hbm, idx_hbm, out_hbm, idx_v, rows_v, gsem, osem):
        wid = lax.axis_index("s") * nc + lax.axis_index("c")
        base = wid * per_b
        pltpu.sync_copy(idx_hbm.at[pl.ds(base, per_b)], idx_v)

        def g_start(r, b):
            for j in range(GRP):
                pltpu.async_copy(table_hbm.at[idx_v.at[r * GRP + j]],
                                 rows_v.at[b, j], gsem.at[b])

        def g_wait(r, b):
            for j in range(GRP):
                pltpu.make_async_copy(table_hbm.at[idx_v.at[r * GRP + j]],
                                      rows_v.at[b, j], gsem.at[b]).wait()

        def o_start(r, b):
            pltpu.async_copy(rows_v.at[b],
                             out_hbm.at[pl.ds(base + r * GRP, GRP)],
                             osem.at[b])

        def o_wait(r, b):
            pltpu.make_async_copy(rows_v.at[b],
                                  out_hbm.at[pl.ds(base + r * GRP, GRP)],
                                  osem.at[b]).wait()

        for r in range(NBUF):
            g_start(r, r % NBUF)
            if r >= LAG:
                g_wait(r - LAG, (r - LAG) % NBUF)
                o_start(r - LAG, (r - LAG) % NBUF)

        @pl.loop(NBUF, rounds, step=NBUF)
        def _(r0):
            for b in range(NBUF):
                r = r0 + b
                o_wait(r - NBUF, b)
                g_start(r, b)
                bl = (b - LAG) % NBUF
                g_wait(r - LAG, bl)
                o_start(r - LAG, bl)

        for i in range(LAG):
            r = rounds - LAG + i
            g_wait(r, r % NBUF)
            o_start(r, r % NBUF)
        for b in range(NBUF):
            o_wait(rounds - NBUF + b, (rounds - NBUF + b) % NBUF)

    return k(table, ids.astype(jnp.int32))


def kernel(player_ids, table):
    outs = []
    off = 0
    for sz in SPLITS:
        outs.append(_sc_gather(lax.slice_in_dim(player_ids, off, off + sz),
                               table))
        off += sz
    return jnp.concatenate(outs, axis=0)


# speedup vs baseline: 1.0330x; 1.0330x over previous
import functools

import jax
import jax.numpy as jnp
from jax import lax
from jax.experimental import pallas as pl
from jax.experimental.pallas import tpu as pltpu
from jax.experimental.pallas import tpu_sc as plsc

GRP = 8
NBUF = 4
LAG = 2
SPLITS = (4096, 4096, 4096, 4096)


def _sc_gather(ids, table):
    batch, hist = ids.shape
    d = table.shape[1]

    info = plsc.get_sparse_core_info()
    nc, ns = info.num_cores, info.num_subcores
    nw = nc * ns
    per_b = batch // nw
    rounds = per_b // GRP

    mesh = plsc.VectorSubcoreMesh(core_axis_name="c", subcore_axis_name="s")

    @functools.partial(
        pl.kernel,
        mesh=mesh,
        compiler_params=pltpu.CompilerParams(use_tc_tiling_on_sc=False),
        out_type=jax.ShapeDtypeStruct((batch, hist, d), table.dtype),
        scratch_types=[
            pltpu.VMEM((per_b, hist), jnp.int32),
            pltpu.VMEM((NBUF, GRP, hist, d), jnp.float32),
            pltpu.SemaphoreType.DMA((NBUF,)),
            pltpu.SemaphoreType.DMA((NBUF,)),
        ],
    )
    def k(table_hbm, idx_hbm, out_hbm, idx_v, rows_v, gsem, osem):
        wid = lax.axis_index("s") * nc + lax.axis_index("c")
        base = wid * per_b
        pltpu.sync_copy(idx_hbm.at[pl.ds(base, per_b)], idx_v)

        def g_start(r, b):
            for j in range(GRP):
                pltpu.async_copy(table_hbm.at[idx_v.at[r * GRP + j]],
                                 rows_v.at[b, j], gsem.at[b])

        def g_wait(r, b):
            for j in range(GRP):
                pltpu.make_async_copy(table_hbm.at[idx_v.at[r * GRP + j]],
                                      rows_v.at[b, j], gsem.at[b]).wait()

        def o_start(r, b):
            pltpu.async_copy(rows_v.at[b],
                             out_hbm.at[pl.ds(base + r * GRP, GRP)],
                             osem.at[b])

        def o_wait(r, b):
            pltpu.make_async_copy(rows_v.at[b],
                                  out_hbm.at[pl.ds(base + r * GRP, GRP)],
                                  osem.at[b]).wait()

        for r in range(NBUF):
            g_start(r, r % NBUF)
            if r >= LAG:
                g_wait(r - LAG, (r - LAG) % NBUF)
                o_start(r - LAG, (r - LAG) % NBUF)

        @pl.loop(NBUF, rounds, step=NBUF)
        def _(r0):
            for b in range(NBUF):
                r = r0 + b
                o_wait(r - NBUF, b)
                g_start(r, b)
                bl = (b - LAG) % NBUF
                g_wait(r - LAG, bl)
                o_start(r - LAG, bl)

        for i in range(LAG):
            r = rounds - LAG + i
            g_wait(r, r % NBUF)
            o_start(r, r % NBUF)
        for b in range(NBUF):
            o_wait(rounds - NBUF + b, (rounds - NBUF + b) % NBUF)

    return k(table, ids.astype(jnp.int32))


def kernel(player_ids, table):
    outs = []
    off = 0
    for sz in SPLITS:
        outs.append(_sc_gather(lax.slice_in_dim(player_ids, off, off + sz),
                               table))
        off += sz
    return jnp.concatenate(outs, axis=0)


# speedup vs baseline: 1.0351x; 1.0020x over previous
import functools

import jax
import jax.numpy as jnp
from jax import lax
from jax.experimental import pallas as pl
from jax.experimental.pallas import tpu as pltpu
from jax.experimental.pallas import tpu_sc as plsc

GRP = 16
NBUF = 4
LAG = 2
SPLITS = (4096, 4096, 4096, 4096)


def _sc_gather(ids, table):
    batch, hist = ids.shape
    d = table.shape[1]

    info = plsc.get_sparse_core_info()
    nc, ns = info.num_cores, info.num_subcores
    nw = nc * ns
    per_b = batch // nw
    rounds = per_b // GRP

    mesh = plsc.VectorSubcoreMesh(core_axis_name="c", subcore_axis_name="s")

    @functools.partial(
        pl.kernel,
        mesh=mesh,
        compiler_params=pltpu.CompilerParams(use_tc_tiling_on_sc=False),
        out_type=jax.ShapeDtypeStruct((batch, hist, d), table.dtype),
        scratch_types=[
            pltpu.VMEM((per_b, hist), jnp.int32),
            pltpu.VMEM((NBUF, GRP, hist, d), jnp.float32),
            pltpu.SemaphoreType.DMA((NBUF,)),
            pltpu.SemaphoreType.DMA((NBUF,)),
        ],
    )
    def k(table_hbm, idx_hbm, out_hbm, idx_v, rows_v, gsem, osem):
        wid = lax.axis_index("s") * nc + lax.axis_index("c")
        base = wid * per_b
        pltpu.sync_copy(idx_hbm.at[pl.ds(base, per_b)], idx_v)

        def g_start(r, b):
            for j in range(GRP):
                pltpu.async_copy(table_hbm.at[idx_v.at[r * GRP + j]],
                                 rows_v.at[b, j], gsem.at[b])

        def g_wait(r, b):
            for j in range(GRP):
                pltpu.make_async_copy(table_hbm.at[idx_v.at[r * GRP + j]],
                                      rows_v.at[b, j], gsem.at[b]).wait()

        def o_start(r, b):
            pltpu.async_copy(rows_v.at[b],
                             out_hbm.at[pl.ds(base + r * GRP, GRP)],
                             osem.at[b])

        def o_wait(r, b):
            pltpu.make_async_copy(rows_v.at[b],
                                  out_hbm.at[pl.ds(base + r * GRP, GRP)],
                                  osem.at[b]).wait()

        for r in range(NBUF):
            g_start(r, r % NBUF)
            if r >= LAG:
                g_wait(r - LAG, (r - LAG) % NBUF)
                o_start(r - LAG, (r - LAG) % NBUF)

        @pl.loop(NBUF, rounds, step=NBUF)
        def _(r0):
            for b in range(NBUF):
                r = r0 + b
                o_wait(r - NBUF, b)
                g_start(r, b)
                bl = (b - LAG) % NBUF
                g_wait(r - LAG, bl)
                o_start(r - LAG, bl)

        for i in range(LAG):
            r = rounds - LAG + i
            g_wait(r, r % NBUF)
            o_start(r, r % NBUF)
        for b in range(NBUF):
            o_wait(rounds - NBUF + b, (rounds - NBUF + b) % NBUF)

    return k(table, ids.astype(jnp.int32))


def kernel(player_ids, table):
    outs = []
    off = 0
    for sz in SPLITS:
        outs.append(_sc_gather(lax.slice_in_dim(player_ids, off, off + sz),
                               table))
        off += sz
    return jnp.concatenate(outs, axis=0)


# speedup vs baseline: 1.0366x; 1.0015x over previous
import functools

import jax
import jax.numpy as jnp
from jax import lax
from jax.experimental import pallas as pl
from jax.experimental.pallas import tpu as pltpu
from jax.experimental.pallas import tpu_sc as plsc

GRP = 16
NBUF = 4
LAG = 3
SPLITS = (4096, 4096, 4096, 4096)


def _sc_gather(ids, table):
    batch, hist = ids.shape
    d = table.shape[1]

    info = plsc.get_sparse_core_info()
    nc, ns = info.num_cores, info.num_subcores
    nw = nc * ns
    per_b = batch // nw
    rounds = per_b // GRP

    mesh = plsc.VectorSubcoreMesh(core_axis_name="c", subcore_axis_name="s")

    @functools.partial(
        pl.kernel,
        mesh=mesh,
        compiler_params=pltpu.CompilerParams(use_tc_tiling_on_sc=False),
        out_type=jax.ShapeDtypeStruct((batch, hist, d), table.dtype),
        scratch_types=[
            pltpu.VMEM((per_b, hist), jnp.int32),
            pltpu.VMEM((NBUF, GRP, hist, d), jnp.float32),
            pltpu.SemaphoreType.DMA((NBUF,)),
            pltpu.SemaphoreType.DMA((NBUF,)),
        ],
    )
    def k(table_hbm, idx_hbm, out_hbm, idx_v, rows_v, gsem, osem):
        wid = lax.axis_index("s") * nc + lax.axis_index("c")
        base = wid * per_b
        pltpu.sync_copy(idx_hbm.at[pl.ds(base, per_b)], idx_v)

        def g_start(r, b):
            for j in range(GRP):
                pltpu.async_copy(table_hbm.at[idx_v.at[r * GRP + j]],
                                 rows_v.at[b, j], gsem.at[b])

        def g_wait(r, b):
            for j in range(GRP):
                pltpu.make_async_copy(table_hbm.at[idx_v.at[r * GRP + j]],
                                      rows_v.at[b, j], gsem.at[b]).wait()

        def o_start(r, b):
            pltpu.async_copy(rows_v.at[b],
                             out_hbm.at[pl.ds(base + r * GRP, GRP)],
                             osem.at[b])

        def o_wait(r, b):
            pltpu.make_async_copy(rows_v.at[b],
                                  out_hbm.at[pl.ds(base + r * GRP, GRP)],
                                  osem.at[b]).wait()

        for r in range(NBUF):
            g_start(r, r % NBUF)
            if r >= LAG:
                g_wait(r - LAG, (r - LAG) % NBUF)
                o_start(r - LAG, (r - LAG) % NBUF)

        @pl.loop(NBUF, rounds, step=NBUF)
        def _(r0):
            for b in range(NBUF):
                r = r0 + b
                o_wait(r - NBUF, b)
                g_start(r, b)
                bl = (b - LAG) % NBUF
                g_wait(r - LAG, bl)
                o_start(r - LAG, bl)

        for i in range(LAG):
            r = rounds - LAG + i
            g_wait(r, r % NBUF)
            o_start(r, r % NBUF)
        for b in range(NBUF):
            o_wait(rounds - NBUF + b, (rounds - NBUF + b) % NBUF)

    return k(table, ids.astype(jnp.int32))


def kernel(player_ids, table):
    outs = []
    off = 0
    for sz in SPLITS:
        outs.append(_sc_gather(lax.slice_in_dim(player_ids, off, off + sz),
                               table))
        off += sz
    return jnp.concatenate(outs, axis=0)
